# Initial kernel scaffold; baseline (speedup 1.0000x reference)
import jax
import jax.numpy as jnp
from jax.experimental import pallas as pl


def kernel(x, edge_index, batch, params):
    raise NotImplementedError("write your pallas kernel here")


# speedup vs baseline: 2.7482x; 2.7482x over previous
import functools

import jax
import jax.numpy as jnp
from jax import lax
from jax.experimental import pallas as pl
from jax.experimental.pallas import tpu as pltpu
from jax.experimental.pallas import tpu_sc as plsc

N = 10000
E = 320000
F = 128
H = 64
NEXP = 4
NCLS = 10
NGRAPH = 128

NC, NS = 2, 16
EPC = E // NC
EPW = EPC // NS
SUB = 80
CHUNK = 400
NSUB = CHUNK // SUB
NCHUNK = EPW // CHUNK
SW = 624
ZR = 24
NZ = SW // ZR
TAIL = N - NS * SW

_f32 = jnp.float32
_i32 = jnp.int32


def _mesh():
    return plsc.VectorSubcoreMesh(core_axis_name="c", subcore_axis_name="s",
                                  num_cores=NC, num_subcores=NS)


def _zero_acc(zbuf, acc, sid, d):
    def zb(i, c):
        for j in range(d // 16):
            zbuf[i, pl.ds(j * 16, 16)] = jnp.zeros((16,), _f32)
        return c

    lax.fori_loop(0, ZR, zb, 0)

    def zc(i, c):
        pltpu.sync_copy(zbuf, acc.at[pl.ds(sid * SW + i * ZR, ZR)])
        return c

    lax.fori_loop(0, NZ, zc, 0)

    @pl.when(sid == NS - 1)
    def _():
        pltpu.sync_copy(zbuf.at[pl.ds(0, TAIL)], acc.at[pl.ds(NS * SW, TAIL)])


def _make_segsum(d, weighted):
    scratch = [
        pltpu.VMEM((NSUB, SUB), _i32),
        pltpu.VMEM((NSUB, SUB), _i32),
        pltpu.VMEM((NSUB, SUB), _i32),
        pltpu.VMEM((NSUB, SUB), _i32),
        pltpu.VMEM((CHUNK,), _f32),
        pltpu.VMEM((CHUNK,), _f32),
        pltpu.VMEM((CHUNK, d), _f32),
        pltpu.VMEM((CHUNK, d), _f32),
        pltpu.VMEM((ZR, d), _f32),
        pltpu.VMEM_SHARED((N, d), _f32),
        pltpu.SemaphoreType.DMA,
        pltpu.SemaphoreType.DMA,
        pltpu.SemaphoreType.DMA,
    ]

    def body(*refs):
        if weighted:
            (h_hbm, src_hbm, dst_hbm, w_hbm, out_hbm, s0, s1, d0, d1,
             w0, w1, r0, r1, zbuf, acc, lsem, gsem, ssem) = refs
        else:
            (h_hbm, src_hbm, dst_hbm, out_hbm, s0, s1, d0, d1,
             w0, w1, r0, r1, zbuf, acc, lsem, gsem, ssem) = refs
        sidx = (s0, s1)
        didx = (d0, d1)
        wbuf = (w0, w1)
        rows = (r0, r1)
        cid = lax.axis_index("c")
        sid = lax.axis_index("s")
        base = cid * EPC + sid * EPW

        _zero_acc(zbuf, acc, sid, d)
        plsc.subcore_barrier()

        def load(g, slot):
            off = base + g * CHUNK
            cps = []
            for j in range(NSUB):
                cps.append(pltpu.async_copy(
                    src_hbm.at[pl.ds(off + j * SUB, SUB)],
                    sidx[slot].at[j], lsem))
                cps.append(pltpu.async_copy(
                    dst_hbm.at[pl.ds(off + j * SUB, SUB)],
                    didx[slot].at[j], lsem))
            if weighted:
                cps.append(pltpu.async_copy(
                    w_hbm.at[pl.ds(off, CHUNK)], wbuf[slot], lsem))
            for c in cps:
                c.wait()

        def fire_gather(slot):
            cps = []
            for j in range(NSUB):
                cps.append(pltpu.async_copy(
                    h_hbm.at[sidx[slot].at[j]],
                    rows[slot].at[pl.ds(j * SUB, SUB)], gsem))
            return cps

        def process(slot):
            if weighted:
                def mul16(t, c):
                    wreg = wbuf[slot][pl.ds(t * 16, 16)]
                    for k in range(16):
                        e = t * 16 + k
                        bc = jnp.full((16,), wreg[k], _f32)
                        for j in range(d // 16):
                            rows[slot][e, pl.ds(j * 16, 16)] = (
                                rows[slot][e, pl.ds(j * 16, 16)] * bc)
                    return c
                lax.fori_loop(0, CHUNK // 16, mul16, 0)
            cps = []
            for j in range(NSUB):
                cps.append(pltpu.async_copy(
                    rows[slot].at[pl.ds(j * SUB, SUB)],
                    acc.at[didx[slot].at[j]], ssem, add=True))
            for c in cps:
                c.wait()

        load(0, 0)
        g0 = fire_gather(0)

        def step(t, carry):
            load(2 * t + 1, 1)
            for c in g0:
                c.wait()
            g1 = fire_gather(1)
            process(0)
            load(2 * t + 2, 0)
            for c in g1:
                c.wait()
            _ = fire_gather(0)
            process(1)
            return carry

        lax.fori_loop(0, (NCHUNK - 1) // 2, step, 0)
        for c in g0:
            c.wait()
        process(0)

        plsc.subcore_barrier()
        pltpu.sync_copy(acc.at[pl.ds(sid * SW, SW)],
                        out_hbm.at[cid, pl.ds(sid * SW, SW)])

        @pl.when(sid == NS - 1)
        def _():
            pltpu.sync_copy(acc.at[pl.ds(NS * SW, TAIL)],
                            out_hbm.at[cid, pl.ds(NS * SW, TAIL)])

    return pl.kernel(
        body,
        out_type=jax.ShapeDtypeStruct((2, N, d), _f32),
        mesh=_mesh(),
        scratch_types=scratch,
        compiler_params=pltpu.CompilerParams(use_tc_tiling_on_sc=False),
    )


def _make_gather_rows(d):
    scratch = [
        pltpu.VMEM((NSUB, SUB), _i32),
        pltpu.VMEM((NSUB, SUB), _i32),
        pltpu.VMEM((CHUNK, d), _f32),
        pltpu.VMEM((CHUNK, d), _f32),
        pltpu.SemaphoreType.DMA,
        pltpu.SemaphoreType.DMA,
    ]

    def body(h_hbm, idx_hbm, out_hbm, s0, s1, r0, r1, lsem, gsem):
        sidx = (s0, s1)
        rows = (r0, r1)
        cid = lax.axis_index("c")
        sid = lax.axis_index("s")
        base = cid * EPC + sid * EPW

        def load(g, slot):
            off = base + g * CHUNK
            cps = [pltpu.async_copy(
                idx_hbm.at[pl.ds(off + j * SUB, SUB)], sidx[slot].at[j], lsem)
                for j in range(NSUB)]
            for c in cps:
                c.wait()

        def fire_gather(slot):
            return [pltpu.async_copy(
                h_hbm.at[sidx[slot].at[j]],
                rows[slot].at[pl.ds(j * SUB, SUB)], gsem)
                for j in range(NSUB)]

        def write_out(g, slot):
            pltpu.sync_copy(rows[slot],
                            out_hbm.at[pl.ds(base + g * CHUNK, CHUNK)])

        load(0, 0)
        g0 = fire_gather(0)

        def step(t, carry):
            load(2 * t + 1, 1)
            for c in g0:
                c.wait()
            g1 = fire_gather(1)
            write_out(2 * t, 0)
            load(2 * t + 2, 0)
            for c in g1:
                c.wait()
            _ = fire_gather(0)
            write_out(2 * t + 1, 1)
            return carry

        lax.fori_loop(0, (NCHUNK - 1) // 2, step, 0)
        for c in g0:
            c.wait()
        write_out(NCHUNK - 1, 0)

    return pl.kernel(
        body,
        out_type=jax.ShapeDtypeStruct((E, d), _f32),
        mesh=_mesh(),
        scratch_types=scratch,
        compiler_params=pltpu.CompilerParams(use_tc_tiling_on_sc=False),
    )


_BN = 1000
_NB = N // _BN
_BE = 2000


def _full(shape):
    return pl.BlockSpec(shape, lambda *_: tuple(0 for _ in shape))


def _gin_mlp_kernel(nagg, h_ref, *refs):
    aggs = refs[:nagg]
    scale_ref, w1_ref, b1_ref, w2_ref, b2_ref, o_ref = refs[nagg:]
    hs = h_ref[...] * scale_ref[0, 0]
    if nagg == 1:
        a = hs + aggs[0][0] + aggs[0][1]
    else:
        a = jnp.concatenate(
            [hs[:, :H] + aggs[0][0] + aggs[0][1],
             hs[:, H:] + aggs[1][0] + aggs[1][1]], axis=1)
    t = jnp.maximum(
        jnp.dot(a, w1_ref[...], preferred_element_type=_f32) + b1_ref[...], 0.0)
    o = jnp.dot(t, w2_ref[...], preferred_element_type=_f32) + b2_ref[...]
    o_ref[...] = jnp.maximum(o, 0.0)


def _gin_layer(h, aggs, p):
    din = h.shape[1]
    scale = (1.0 + p["eps"]).reshape(1, 1).astype(_f32)
    return pl.pallas_call(
        functools.partial(_gin_mlp_kernel, len(aggs)),
        grid=(_NB,),
        in_specs=[
            pl.BlockSpec((_BN, din), lambda i: (i, 0)),
        ] + [
            pl.BlockSpec((2, _BN, H), lambda i: (0, i, 0)) for _ in aggs
        ] + [
            _full((1, 1)),
            _full((din, H)),
            _full((1, H)),
            _full((H, H)),
            _full((1, H)),
        ],
        out_specs=pl.BlockSpec((_BN, H), lambda i: (i, 0)),
        out_shape=jax.ShapeDtypeStruct((N, H), _f32),
    )(h, *aggs, scale, p["W1"], p["b1"].reshape(1, H), p["W2"],
      p["b2"].reshape(1, H))


def _gin_mlp4_kernel(nagg, h_ref, *refs):
    aggs = refs[:nagg]
    scale_ref, w1_ref, b1_ref, w2_ref, b2_ref, o_ref = refs[nagg:]
    hs = h_ref[0] * scale_ref[0, 0]
    if nagg == 1:
        a = hs + aggs[0][0, 0] + aggs[0][0, 1]
    else:
        a = jnp.concatenate(
            [hs[:, :H] + aggs[0][0, 0] + aggs[0][0, 1],
             hs[:, H:] + aggs[1][0, 0] + aggs[1][0, 1]], axis=1)
    t = jnp.maximum(
        jnp.dot(a, w1_ref[...], preferred_element_type=_f32) + b1_ref[...], 0.0)
    o = jnp.dot(t, w2_ref[...], preferred_element_type=_f32) + b2_ref[...]
    o_ref[0] = jnp.maximum(o, 0.0)


def _gin_layer4(hc, agg4s, p):
    din = hc.shape[2]
    scale = (1.0 + p["eps"]).reshape(1, 1).astype(_f32)
    return pl.pallas_call(
        functools.partial(_gin_mlp4_kernel, len(agg4s)),
        grid=(NEXP, _NB),
        in_specs=[
            pl.BlockSpec((1, _BN, din), lambda e, i: (e, i, 0)),
        ] + [
            pl.BlockSpec((1, 2, _BN, H), lambda e, i: (e, 0, i, 0))
            for _ in agg4s
        ] + [
            _full((1, 1)),
            _full((din, H)),
            _full((1, H)),
            _full((H, H)),
            _full((1, H)),
        ],
        out_specs=pl.BlockSpec((1, _BN, H), lambda e, i: (e, i, 0)),
        out_shape=jax.ShapeDtypeStruct((NEXP, N, H), _f32),
    )(hc, *agg4s, scale, p["W1"], p["b1"].reshape(1, H), p["W2"],
      p["b2"].reshape(1, H))


def _prep_kernel(x_ref, h_ref, nw1, nb1, nw2, nb2, fw1, fb1, fw2, fb2,
                 xm_ref, nm_ref):
    h = h_ref[...]
    x = x_ref[...]
    for e in range(NEXP):
        t = jnp.maximum(
            jnp.dot(h, nw1[e], preferred_element_type=_f32) + nb1[e], 0.0)
        nm = jax.nn.sigmoid(
            jnp.dot(t, nw2[e], preferred_element_type=_f32) + nb2[e])
        t2 = jnp.maximum(
            jnp.dot(h, fw1[e], preferred_element_type=_f32) + fb1[e], 0.0)
        fm = jax.nn.sigmoid(
            jnp.dot(t2, fw2[e], preferred_element_type=_f32) + fb2[e])
        xm_ref[e] = x * fm * nm
        nm_ref[:, e:e + 1] = nm


def _prep(x, h, nmps, fmps):
    nw1 = jnp.stack([p["W1"] for p in nmps])
    nb1 = jnp.stack([p["b1"].reshape(1, H) for p in nmps])
    nw2 = jnp.stack([p["W2"] for p in nmps])
    nb2 = jnp.stack([p["b2"].reshape(1, 1) for p in nmps])
    fw1 = jnp.stack([p["W1"] for p in fmps])
    fb1 = jnp.stack([p["b1"].reshape(1, H) for p in fmps])
    fw2 = jnp.stack([p["W2"] for p in fmps])
    fb2 = jnp.stack([p["b2"].reshape(1, F) for p in fmps])
    return pl.pallas_call(
        _prep_kernel,
        grid=(_NB,),
        in_specs=[
            pl.BlockSpec((_BN, F), lambda i: (i, 0)),
            pl.BlockSpec((_BN, H), lambda i: (i, 0)),
            _full((NEXP, H, H)), _full((NEXP, 1, H)),
            _full((NEXP, H, 1)), _full((NEXP, 1, 1)),
            _full((NEXP, H, H)), _full((NEXP, 1, H)),
            _full((NEXP, H, F)), _full((NEXP, 1, F)),
        ],
        out_specs=[
            pl.BlockSpec((NEXP, _BN, F), lambda i: (0, i, 0)),
            pl.BlockSpec((_BN, NEXP), lambda i: (i, 0)),
        ],
        out_shape=[
            jax.ShapeDtypeStruct((NEXP, N, F), _f32),
            jax.ShapeDtypeStruct((N, NEXP), _f32),
        ],
    )(x, h, nw1, nb1, nw2, nb2, fw1, fb1, fw2, fb2)


def _edge_mlp_kernel(hs_ref, hd_ref, w1a, w1b, b1, w2, b2, o_ref):
    hs = hs_ref[...]
    hd = hd_ref[...]
    for e in range(NEXP):
        z = jnp.maximum(
            jnp.dot(hs, w1a[e], preferred_element_type=_f32)
            + jnp.dot(hd, w1b[e], preferred_element_type=_f32) + b1[e], 0.0)
        lg = jnp.dot(z, w2[e], preferred_element_type=_f32) + b2[e]
        o_ref[:, e:e + 1] = jax.nn.sigmoid(lg)


def _edge_mlp(hsrc, hdst, emps):
    w1a = jnp.stack([p["W1"][:H] for p in emps])
    w1b = jnp.stack([p["W1"][H:] for p in emps])
    b1 = jnp.stack([p["b1"].reshape(1, H) for p in emps])
    w2 = jnp.stack([p["W2"] for p in emps])
    b2 = jnp.stack([p["b2"].reshape(1, 1) for p in emps])
    return pl.pallas_call(
        _edge_mlp_kernel,
        grid=(E // _BE,),
        in_specs=[
            pl.BlockSpec((_BE, H), lambda i: (i, 0)),
            pl.BlockSpec((_BE, H), lambda i: (i, 0)),
            _full((NEXP, H, H)), _full((NEXP, H, H)), _full((NEXP, 1, H)),
            _full((NEXP, H, 1)), _full((NEXP, 1, 1)),
        ],
        out_specs=pl.BlockSpec((_BE, NEXP), lambda i: (i, 0)),
        out_shape=jax.ShapeDtypeStruct((E, NEXP), _f32),
    )(hsrc, hdst, w1a, w1b, b1, w2, b2)


def _pool_kernel(hc_ref, nm_ref, b_ref, cw_ref, cb_ref, o_ref, gacc, cacc):
    i = pl.program_id(0)

    @pl.when(i == 0)
    def _():
        gacc[...] = jnp.zeros((NEXP, NGRAPH, H), _f32)
        cacc[...] = jnp.zeros((NGRAPH, NEXP), _f32)

    bb = b_ref[0, 0]
    oh = (bb[None, :] == lax.broadcasted_iota(_i32, (NGRAPH, _BN), 0)
          ).astype(_f32)
    nm = nm_ref[...]
    cacc[...] += jnp.dot(oh, nm, preferred_element_type=_f32)
    for e in range(NEXP):
        gacc[e] += jnp.dot(oh, hc_ref[e] * nm[:, e:e + 1],
                           preferred_element_type=_f32)

    @pl.when(i == _NB - 1)
    def _():
        for e in range(NEXP):
            g = gacc[e] / jnp.maximum(cacc[...][:, e:e + 1], 1e-6)
            o_ref[e] = (jnp.dot(g, cw_ref[e], preferred_element_type=_f32)
                        + cb_ref[e])


def _pool_clf(hc, nm4, batch, clfps):
    cw = jnp.stack([p["W"] for p in clfps])
    cb = jnp.stack([p["b"].reshape(1, NCLS) for p in clfps])
    b3 = batch.reshape(_NB, 1, _BN).astype(_i32)
    return pl.pallas_call(
        _pool_kernel,
        grid=(_NB,),
        in_specs=[
            pl.BlockSpec((NEXP, _BN, H), lambda i: (0, i, 0)),
            pl.BlockSpec((_BN, NEXP), lambda i: (i, 0)),
            pl.BlockSpec((1, 1, _BN), lambda i: (i, 0, 0)),
            _full((NEXP, H, NCLS)), _full((NEXP, 1, NCLS)),
        ],
        out_specs=pl.BlockSpec((NEXP, NGRAPH, NCLS), lambda i: (0, 0, 0)),
        out_shape=jax.ShapeDtypeStruct((NEXP, NGRAPH, NCLS), _f32),
        scratch_shapes=[
            pltpu.VMEM((NEXP, NGRAPH, H), _f32),
            pltpu.VMEM((NGRAPH, NEXP), _f32),
        ],
    )(hc, nm4, b3, cw, cb)


_make_segsum = functools.lru_cache(None)(_make_segsum)
_make_gather_rows = functools.lru_cache(None)(_make_gather_rows)


def _segsum_w(h, src, dst, w):
    return _make_segsum(h.shape[1], True)(h, src, dst, w)


def _gather64(h, idx):
    return _make_gather_rows(H)(h, idx)


def _segsum_cols(h, src, dst, w):
    if h.shape[1] == H:
        return [_segsum_w(h, src, dst, w)]
    return [_segsum_w(h[:, :H], src, dst, w),
            _segsum_w(h[:, H:], src, dst, w)]


def kernel(x, edge_index, batch, params):
    src = edge_index[0]
    dst = edge_index[1]

    h = x
    ones_e = jnp.ones((E,), _f32)
    for p in params["causal"]:
        aggs = _segsum_cols(h, src, dst, ones_e)
        h = _gin_layer(h, aggs, p)

    xm4, nm4 = _prep(x, h, params["node_mask"], params["feat_mask"])

    hsrc = _gather64(h, src)
    hdst = _gather64(h, dst)
    em4 = _edge_mlp(hsrc, hdst, params["edge_mask"]).T

    hc = xm4
    for p in params["clf_enc"]:
        agg4s = []
        nhalf = hc.shape[2] // H
        for half in range(nhalf):
            part = hc[:, :, half * H:(half + 1) * H]
            agg4s.append(jnp.stack(
                [_segsum_w(part[e], src, dst, em4[e]) for e in range(NEXP)]))
        hc = _gin_layer4(hc, agg4s, p)

    return _pool_clf(hc, nm4, batch, params["clf"])
